# baseline (speedup 1.0000x reference)
import functools

import jax
import jax.numpy as jnp
from jax import lax
from jax.experimental import pallas as pl
from jax.experimental.pallas import tpu as pltpu
from jax.experimental.pallas import tpu_sc as plsc

N = 10000
E = 160000
D = 256
ENC = 32
NG = 16
H = 64
DEPTH = 4
HID = 256
OUT = 128
NUM_ENC = 3

CHUNK = 128
EPAD = 163840
NCHUNK_T = 80
SLAB = 40
NBUF = 4
ROWS_T = 640
ACC_ROWS = 10240
WBLK = 80

_SC_PARAMS = pltpu.CompilerParams(use_tc_tiling_on_sc=False)


def _spmm_kernel(Fh):
    mesh = plsc.VectorSubcoreMesh(core_axis_name="c", subcore_axis_name="s")

    @functools.partial(
        pl.kernel,
        out_type=jax.ShapeDtypeStruct((2, ACC_ROWS, Fh), jnp.float32),
        mesh=mesh,
        scratch_types=[
            pltpu.VMEM_SHARED((ACC_ROWS, Fh), jnp.float32),
            pltpu.VMEM((SLAB, CHUNK), jnp.int32),
            pltpu.VMEM((SLAB, CHUNK), jnp.int32),
        ] + [pltpu.VMEM((CHUNK, Fh), jnp.float32)] * NBUF
          + [pltpu.SemaphoreType.DMA] * (2 * NBUF),
        compiler_params=_SC_PARAMS,
    )
    def k(inq, idx, dstl, zrows, out, acc, idxv, dstv,
          gb0, gb1, gb2, gb3, sg0, sg1, sg2, sg3, ss0, ss1, ss2, ss3):
        gbs = (gb0, gb1, gb2, gb3)
        sgs = (sg0, sg1, sg2, sg3)
        sss = (ss0, ss1, ss2, ss3)
        c = lax.axis_index("c")
        s = lax.axis_index("s")
        pltpu.sync_copy(zrows, acc.at[pl.ds(s * ROWS_T, ROWS_T)])
        plsc.subcore_barrier()

        def slab_body(sl, carry):
            pltpu.sync_copy(idx.at[c, s, pl.ds(sl * SLAB, SLAB)], idxv)
            pltpu.sync_copy(dstl.at[s, pl.ds(sl * SLAB, SLAB)], dstv)
            for b in range(NBUF):
                pltpu.async_copy(inq.at[idxv.at[b]], gbs[b], sgs[b])

            def grp(m, carry2):
                for b in range(NBUF):
                    pltpu.make_async_copy(inq.at[pl.ds(0, CHUNK)], gbs[b],
                                          sgs[b]).wait()
                    pltpu.async_copy(gbs[b], acc.at[dstv.at[NBUF * m + b]],
                                     sss[b], add=True)
                for b in range(NBUF):
                    pltpu.make_async_copy(inq.at[pl.ds(0, CHUNK)], gbs[b],
                                          sss[b]).wait()

                    @pl.when(m < SLAB // NBUF - 1)
                    def _(b=b):
                        pltpu.async_copy(
                            inq.at[idxv.at[NBUF * (m + 1) + b]], gbs[b],
                            sgs[b])

                return carry2

            lax.fori_loop(0, SLAB // NBUF, grp, 0)
            return carry

        lax.fori_loop(0, NCHUNK_T // SLAB, slab_body, 0)
        plsc.subcore_barrier()

        def wstep(b, carry):
            row0 = s * ROWS_T + b * WBLK
            pltpu.sync_copy(acc.at[pl.ds(row0, WBLK)],
                            out.at[c, pl.ds(row0, WBLK)])
            return carry

        lax.fori_loop(0, ROWS_T // WBLK, wstep, 0)

    return k


def _spmm_es_kernel(W):
    mesh = plsc.VectorSubcoreMesh(core_axis_name="c", subcore_axis_name="s")
    nch = NCHUNK_T // 2

    @functools.partial(
        pl.kernel,
        out_type=jax.ShapeDtypeStruct((2, ACC_ROWS, W), jnp.float32),
        mesh=mesh,
        scratch_types=[
            pltpu.VMEM_SHARED((ACC_ROWS, W), jnp.float32),
            pltpu.VMEM((nch, CHUNK), jnp.int32),
            pltpu.VMEM((nch, CHUNK), jnp.int32),
        ] + [pltpu.VMEM((CHUNK, W), jnp.float32)] * NBUF
          + [pltpu.SemaphoreType.DMA] * (2 * NBUF),
        compiler_params=_SC_PARAMS,
    )
    def k(inq, idx, dstl, zrows, out, acc, idxv, dstv,
          gb0, gb1, gb2, gb3, sg0, sg1, sg2, sg3, ss0, ss1, ss2, ss3):
        gbs = (gb0, gb1, gb2, gb3)
        sgs = (sg0, sg1, sg2, sg3)
        sss = (ss0, ss1, ss2, ss3)
        c = lax.axis_index("c")
        s = lax.axis_index("s")
        pltpu.sync_copy(idx.at[c, s], idxv)
        pltpu.sync_copy(dstl.at[c, s], dstv)
        pltpu.sync_copy(zrows, acc.at[pl.ds(s * ROWS_T, ROWS_T)])
        plsc.subcore_barrier()

        for b in range(NBUF):
            pltpu.async_copy(inq.at[idxv.at[b]], gbs[b], sgs[b])

        def grp(m, carry):
            for b in range(NBUF):
                pltpu.make_async_copy(inq.at[pl.ds(0, CHUNK)], gbs[b],
                                      sgs[b]).wait()
                pltpu.async_copy(gbs[b], acc.at[dstv.at[NBUF * m + b]],
                                 sss[b], add=True)
            for b in range(NBUF):
                pltpu.make_async_copy(inq.at[pl.ds(0, CHUNK)], gbs[b],
                                      sss[b]).wait()

                @pl.when(m < nch // NBUF - 1)
                def _(b=b):
                    pltpu.async_copy(inq.at[idxv.at[NBUF * (m + 1) + b]],
                                     gbs[b], sgs[b])

            return carry

        lax.fori_loop(0, nch // NBUF, grp, 0)
        plsc.subcore_barrier()

        def wstep(b, carry):
            row0 = s * ROWS_T + b * WBLK
            pltpu.sync_copy(acc.at[pl.ds(row0, WBLK)],
                            out.at[c, pl.ds(row0, WBLK)])
            return carry

        lax.fori_loop(0, ROWS_T // WBLK, wstep, 0)

    return k


def _deg_kernel():
    mesh = plsc.VectorSubcoreMesh(core_axis_name="c", subcore_axis_name="s")

    @functools.partial(
        pl.kernel,
        out_type=jax.ShapeDtypeStruct((2, ACC_ROWS, 16), jnp.float32),
        mesh=mesh,
        scratch_types=[
            pltpu.VMEM_SHARED((ACC_ROWS, 16), jnp.float32),
            pltpu.VMEM((NCHUNK_T // 2, CHUNK), jnp.int32),
            pltpu.VMEM((CHUNK, 16), jnp.float32),
        ] + [pltpu.SemaphoreType.DMA] * NBUF,
        compiler_params=_SC_PARAMS,
    )
    def k(dstl, ones_h, zrows, out, acc, dstv, onesv, ss0, ss1, ss2, ss3):
        sss = (ss0, ss1, ss2, ss3)
        c = lax.axis_index("c")
        s = lax.axis_index("s")
        pltpu.sync_copy(dstl.at[c, s], dstv)
        pltpu.sync_copy(ones_h, onesv)
        pltpu.sync_copy(zrows, acc.at[pl.ds(s * ROWS_T, ROWS_T)])
        plsc.subcore_barrier()

        for b in range(NBUF):
            pltpu.async_copy(onesv, acc.at[dstv.at[b]], sss[b], add=True)

        nch = NCHUNK_T // 2

        def grp(m, carry):
            for b in range(NBUF):
                pltpu.make_async_copy(ones_h, onesv, sss[b]).wait()

                @pl.when(m < nch // NBUF - 1)
                def _(b=b):
                    pltpu.async_copy(onesv, acc.at[dstv.at[NBUF * (m + 1) + b]],
                                     sss[b], add=True)

            return carry

        lax.fori_loop(0, nch // NBUF, grp, 0)
        plsc.subcore_barrier()

        def wstep(b, carry):
            row0 = s * ROWS_T + b * WBLK
            pltpu.sync_copy(acc.at[pl.ds(row0, WBLK)],
                            out.at[c, pl.ds(row0, WBLK)])
            return carry

        lax.fori_loop(0, ROWS_T // WBLK, wstep, 0)

    return k


def _norm_of(degblk):
    cnt = degblk[0, :, 0:1] + degblk[1, :, 0:1]
    return lax.rsqrt(cnt + 1.0)


NB = 2000


def _idx_body(src_ref, o0_ref, o1_ref, o2_ref, o3_ref):
    src = src_ref[...]
    cofs = lax.broadcasted_iota(jnp.int32, (2, EPAD), 0) * N
    for l, o_ref in enumerate((o0_ref, o1_ref, o2_ref, o3_ref)):
        o_ref[...] = src + cofs + (2 * l * N)


def _router_in_body(x_ref, w_ref, b_ref, o_ref):
    o_ref[...] = jax.nn.relu(
        jnp.dot(x_ref[...], w_ref[...], preferred_element_type=jnp.float32)
        + b_ref[...])


def _gin_body(h_ref, agg_ref, w1_ref, b1_ref, w2_ref, b2_ref, o_ref):
    h = h_ref[...] + agg_ref[0] + agg_ref[1]
    h = jax.nn.relu(jnp.dot(h, w1_ref[...], preferred_element_type=jnp.float32)
                    + b1_ref[...])
    o_ref[...] = jax.nn.relu(
        jnp.dot(h, w2_ref[...], preferred_element_type=jnp.float32)
        + b2_ref[...])


def _router_head_body(h_ref, batch_ref, wo_ref, bo_ref, o_ref):
    onehot = (lax.broadcasted_iota(jnp.int32, (N, NG), 1)
              == batch_ref[...]).astype(jnp.float32)
    pooled = lax.dot_general(onehot, h_ref[...], (((0,), (0,)), ((), ())),
                             preferred_element_type=jnp.float32)
    counts = jnp.maximum(jnp.sum(onehot, axis=0, keepdims=True), 1.0).T
    logits = (jnp.dot(pooled / counts, wo_ref[...],
                      preferred_element_type=jnp.float32) + bo_ref[...])
    m = jnp.max(logits, axis=-1, keepdims=True)
    e = jnp.exp(logits - m)
    w = e / jnp.sum(e, axis=-1, keepdims=True)
    o_ref[...] = w / counts


def _prep_u_body(x_ref, f0_ref, f1_ref, f2_ref, deg_ref, o_ref):
    nrm = _norm_of(deg_ref[...])
    u = jnp.concatenate(
        [x_ref[...], f0_ref[:, D:], f1_ref[:, D:], f2_ref[:, D:],
         jnp.zeros((x_ref.shape[0], 32), jnp.float32)], axis=-1) * nrm
    for q in range(4):
        o_ref[q] = u[:, 96 * q:96 * (q + 1)]


def _z1_body(p0_ref, p1_ref, u_ref, deg_ref, w1_ref, b1_ref, o_ref):
    nrm = _norm_of(deg_ref[...])
    pcat = jnp.concatenate([p0_ref[0], p0_ref[1], p1_ref[0], p1_ref[1]],
                           axis=-1)
    ucat = jnp.concatenate([u_ref[q] for q in range(4)], axis=-1)
    sfull = nrm * (pcat + ucat)
    sx = sfull[:, :D]
    sf = sfull[:, D:D + 3 * ENC]
    shared = jnp.dot(sx, w1_ref[:D], preferred_element_type=jnp.float32)
    cols = []
    for i in range(NUM_ENC):
        z = jax.nn.relu(
            shared
            + jnp.dot(sf[:, ENC * i:ENC * (i + 1)], w1_ref[D:],
                      preferred_element_type=jnp.float32)
            + b1_ref[...])
        cols.append(z * nrm)
    uz = jnp.concatenate(cols, axis=-1)
    for q in range(8):
        o_ref[q] = uz[:, 96 * q:96 * (q + 1)]


def _z2_body(a0_ref, a1_ref, a2_ref, a3_ref, u_ref, deg_ref, w2_ref, b2_ref,
             w3_ref, o_ref):
    nrm = _norm_of(deg_ref[...])
    acat = jnp.concatenate(
        [a0_ref[0], a0_ref[1], a1_ref[0], a1_ref[1],
         a2_ref[0], a2_ref[1], a3_ref[0], a3_ref[1]], axis=-1)
    ucat = jnp.concatenate([u_ref[q] for q in range(8)], axis=-1)
    cols = []
    for i in range(NUM_ENC):
        sfull = nrm * (acat[:, HID * i:HID * (i + 1)]
                       + ucat[:, HID * i:HID * (i + 1)])
        t = jax.nn.relu(
            jnp.dot(sfull, w2_ref[...], preferred_element_type=jnp.float32)
            + b2_ref[...])
        y = jnp.dot(t, w3_ref[...], preferred_element_type=jnp.float32)
        cols.append(y * nrm)
    ycat = jnp.concatenate(cols, axis=-1)
    for q in range(4):
        o_ref[q] = ycat[:, 96 * q:96 * (q + 1)]


def _final_body(a0_ref, a1_ref, y_ref, deg_ref, b3_ref, batch_ref,
                wc_ref, o_ref, acc_ref):
    i = pl.program_id(0)

    @pl.when(i == 0)
    def _():
        acc_ref[...] = jnp.zeros_like(acc_ref)

    nrm = _norm_of(deg_ref[...])
    onehot = (lax.broadcasted_iota(jnp.int32, (NB, NG), 1)
              == batch_ref[...]).astype(jnp.float32)
    acat = jnp.concatenate([a0_ref[0], a0_ref[1], a1_ref[0], a1_ref[1]],
                           axis=-1)
    ycat = jnp.concatenate([y_ref[q] for q in range(4)], axis=-1)
    for e in range(NUM_ENC):
        z3 = jax.nn.relu(nrm * (acat[:, OUT * e:OUT * (e + 1)]
                                + ycat[:, OUT * e:OUT * (e + 1)])
                         + b3_ref[...])
        acc_ref[:, OUT * e:OUT * (e + 1)] += lax.dot_general(
            onehot, z3, (((0,), (0,)), ((), ())),
            preferred_element_type=jnp.float32)

    @pl.when(i == pl.num_programs(0) - 1)
    def _():
        o_ref[...] = sum(
            wc_ref[:, e:e + 1] * acc_ref[:, OUT * e:OUT * (e + 1)]
            for e in range(NUM_ENC))


def _row_grid(nblk):
    return (N // nblk,)


def kernel(x, edge_index, batch, enc0, enc1, enc2, Wr_in, br_in, Wg1, bg1,
           Wg2, bg2, Wr_out, br_out, W1, b1, W2, b2, W3, b3):
    src = edge_index[0]
    dst = edge_index[1]
    npad = EPAD - E
    src_pad = jnp.concatenate([src, jnp.zeros((npad,), jnp.int32)])
    dst_pad = jnp.concatenate([dst, jnp.full((npad,), N, jnp.int32)])

    idxp = pl.pallas_call(
        _idx_body,
        in_specs=[pl.BlockSpec((1, EPAD), lambda: (0, 0))],
        out_specs=[pl.BlockSpec((2, EPAD), lambda: (0, 0))] * 4,
        out_shape=[jax.ShapeDtypeStruct((2, EPAD), jnp.int32)] * 4,
    )(src_pad.reshape(1, EPAD))
    idxp = [a.reshape(2, 16, NCHUNK_T, CHUNK) for a in idxp]
    src232 = src_pad.reshape(2, 16, NCHUNK_T // 2, CHUNK)
    dst16 = dst_pad.reshape(16, NCHUNK_T, CHUNK)
    dst232 = dst_pad.reshape(2, 16, NCHUNK_T // 2, CHUNK)

    z16 = jnp.zeros((ROWS_T, 16), jnp.float32)
    z64 = jnp.zeros((ROWS_T, 64), jnp.float32)
    z96 = jnp.zeros((ROWS_T, 96), jnp.float32)
    ones16 = jnp.ones((CHUNK, 16), jnp.float32)

    deg = _deg_kernel()(dst232, ones16, z16)

    h = pl.pallas_call(
        _router_in_body,
        grid=_row_grid(NB),
        in_specs=[
            pl.BlockSpec((NB, D), lambda i: (i, 0)),
            pl.BlockSpec((D, H), lambda i: (0, 0)),
            pl.BlockSpec((1, H), lambda i: (0, 0)),
        ],
        out_specs=pl.BlockSpec((NB, H), lambda i: (i, 0)),
        out_shape=jax.ShapeDtypeStruct((N, H), jnp.float32),
    )(x, Wr_in, br_in.reshape(1, H))

    spmm64 = _spmm_es_kernel(64)
    for l in range(DEPTH):
        agg = spmm64(h, src232, dst232, z64)
        h = pl.pallas_call(
            _gin_body,
            grid=_row_grid(NB),
            in_specs=[
                pl.BlockSpec((NB, H), lambda i: (i, 0)),
                pl.BlockSpec((2, NB, H), lambda i: (0, i, 0)),
                pl.BlockSpec((H, H), lambda i: (0, 0)),
                pl.BlockSpec((1, H), lambda i: (0, 0)),
                pl.BlockSpec((H, H), lambda i: (0, 0)),
                pl.BlockSpec((1, H), lambda i: (0, 0)),
            ],
            out_specs=pl.BlockSpec((NB, H), lambda i: (i, 0)),
            out_shape=jax.ShapeDtypeStruct((N, H), jnp.float32),
        )(h, agg, Wg1[l], bg1[l].reshape(1, H), Wg2[l], bg2[l].reshape(1, H))

    batch_col = batch.astype(jnp.int32).reshape(N, 1)
    wc = pl.pallas_call(
        _router_head_body,
        in_specs=[
            pl.BlockSpec((N, H), lambda: (0, 0)),
            pl.BlockSpec((N, 1), lambda: (0, 0)),
            pl.BlockSpec((H, NUM_ENC), lambda: (0, 0)),
            pl.BlockSpec((1, NUM_ENC), lambda: (0, 0)),
        ],
        out_specs=pl.BlockSpec((NG, NUM_ENC), lambda: (0, 0)),
        out_shape=jax.ShapeDtypeStruct((NG, NUM_ENC), jnp.float32),
    )(h, batch_col, Wr_out, br_out.reshape(1, NUM_ENC))

    u = pl.pallas_call(
        _prep_u_body,
        grid=_row_grid(NB),
        in_specs=[
            pl.BlockSpec((NB, D), lambda i: (i, 0)),
            pl.BlockSpec((NB, D + ENC), lambda i: (i, 0)),
            pl.BlockSpec((NB, D + ENC), lambda i: (i, 0)),
            pl.BlockSpec((NB, D + ENC), lambda i: (i, 0)),
            pl.BlockSpec((2, NB, 16), lambda i: (0, i, 0)),
        ],
        out_specs=pl.BlockSpec((4, NB, 96), lambda i: (0, i, 0)),
        out_shape=jax.ShapeDtypeStruct((4, N, 96), jnp.float32),
    )(x, enc0, enc1, enc2, deg)

    spmm96 = _spmm_kernel(96)
    u4 = u.reshape(4 * N, 96)
    p = [spmm96(u4, idxp[i], dst16, z96) for i in range(2)]

    uz = pl.pallas_call(
        _z1_body,
        grid=_row_grid(NB),
        in_specs=[
            pl.BlockSpec((2, NB, 96), lambda i: (0, i, 0)),
            pl.BlockSpec((2, NB, 96), lambda i: (0, i, 0)),
            pl.BlockSpec((4, NB, 96), lambda i: (0, i, 0)),
            pl.BlockSpec((2, NB, 16), lambda i: (0, i, 0)),
            pl.BlockSpec((D + ENC, HID), lambda i: (0, 0)),
            pl.BlockSpec((1, HID), lambda i: (0, 0)),
        ],
        out_specs=pl.BlockSpec((8, NB, 96), lambda i: (0, i, 0)),
        out_shape=jax.ShapeDtypeStruct((8, N, 96), jnp.float32),
    )(p[0], p[1], u, deg, W1, b1.reshape(1, HID))

    uz8 = uz.reshape(8 * N, 96)
    a2 = [spmm96(uz8, idxp[i], dst16, z96) for i in range(4)]

    y = pl.pallas_call(
        _z2_body,
        grid=_row_grid(NB),
        in_specs=[
            pl.BlockSpec((2, NB, 96), lambda i: (0, i, 0)),
            pl.BlockSpec((2, NB, 96), lambda i: (0, i, 0)),
            pl.BlockSpec((2, NB, 96), lambda i: (0, i, 0)),
            pl.BlockSpec((2, NB, 96), lambda i: (0, i, 0)),
            pl.BlockSpec((8, NB, 96), lambda i: (0, i, 0)),
            pl.BlockSpec((2, NB, 16), lambda i: (0, i, 0)),
            pl.BlockSpec((HID, HID), lambda i: (0, 0)),
            pl.BlockSpec((1, HID), lambda i: (0, 0)),
            pl.BlockSpec((HID, OUT), lambda i: (0, 0)),
        ],
        out_specs=pl.BlockSpec((4, NB, 96), lambda i: (0, i, 0)),
        out_shape=jax.ShapeDtypeStruct((4, N, 96), jnp.float32),
    )(a2[0], a2[1], a2[2], a2[3], uz, deg, W2, b2.reshape(1, HID), W3)

    y4 = y.reshape(4 * N, 96)
    a3 = [spmm96(y4, idxp[i], dst16, z96) for i in range(2)]

    final = pl.pallas_call(
        _final_body,
        grid=_row_grid(NB),
        in_specs=[
            pl.BlockSpec((2, NB, 96), lambda i: (0, i, 0)),
            pl.BlockSpec((2, NB, 96), lambda i: (0, i, 0)),
            pl.BlockSpec((4, NB, 96), lambda i: (0, i, 0)),
            pl.BlockSpec((2, NB, 16), lambda i: (0, i, 0)),
            pl.BlockSpec((1, OUT), lambda i: (0, 0)),
            pl.BlockSpec((NB, 1), lambda i: (i, 0)),
            pl.BlockSpec((NG, NUM_ENC), lambda i: (0, 0)),
        ],
        out_specs=pl.BlockSpec((NG, OUT), lambda i: (0, 0)),
        out_shape=jax.ShapeDtypeStruct((NG, OUT), jnp.float32),
        scratch_shapes=[pltpu.VMEM((NG, 3 * OUT), jnp.float32)],
    )(a3[0], a3[1], y, deg, b3.reshape(1, OUT), batch_col, wc)

    return final


# speedup vs baseline: 1.0238x; 1.0238x over previous
import functools

import jax
import jax.numpy as jnp
from jax import lax
from jax.experimental import pallas as pl
from jax.experimental.pallas import tpu as pltpu
from jax.experimental.pallas import tpu_sc as plsc

N = 10000
E = 160000
D = 256
ENC = 32
NG = 16
H = 64
DEPTH = 4
HID = 256
OUT = 128
NUM_ENC = 3

CHUNK = 128
EPAD = 163840
NCHUNK_T = 80
SLAB = 40
NBUF = 4
ROWS_T = 640
ACC_ROWS = 10240
WBLK = 80

_SC_PARAMS = pltpu.CompilerParams(use_tc_tiling_on_sc=False)


def _spmm_kernel(Fh):
    mesh = plsc.VectorSubcoreMesh(core_axis_name="c", subcore_axis_name="s")

    @functools.partial(
        pl.kernel,
        out_type=jax.ShapeDtypeStruct((2, ACC_ROWS, Fh), jnp.float32),
        mesh=mesh,
        scratch_types=[
            pltpu.VMEM_SHARED((ACC_ROWS, Fh), jnp.float32),
            pltpu.VMEM((SLAB, CHUNK), jnp.int32),
            pltpu.VMEM((SLAB, CHUNK), jnp.int32),
        ] + [pltpu.VMEM((CHUNK, Fh), jnp.float32)] * NBUF
          + [pltpu.SemaphoreType.DMA] * (2 * NBUF),
        compiler_params=_SC_PARAMS,
    )
    def k(inq, idx, dstl, zrows, out, acc, idxv, dstv,
          gb0, gb1, gb2, gb3, sg0, sg1, sg2, sg3, ss0, ss1, ss2, ss3):
        gbs = (gb0, gb1, gb2, gb3)
        sgs = (sg0, sg1, sg2, sg3)
        sss = (ss0, ss1, ss2, ss3)
        c = lax.axis_index("c")
        s = lax.axis_index("s")
        pltpu.sync_copy(zrows, acc.at[pl.ds(s * ROWS_T, ROWS_T)])
        plsc.subcore_barrier()

        def slab_body(sl, carry):
            pltpu.sync_copy(idx.at[c, s, pl.ds(sl * SLAB, SLAB)], idxv)
            pltpu.sync_copy(dstl.at[s, pl.ds(sl * SLAB, SLAB)], dstv)
            for b in range(NBUF):
                pltpu.async_copy(inq.at[idxv.at[b]], gbs[b], sgs[b])

            def grp(m, carry2):
                for b in range(NBUF):
                    pltpu.make_async_copy(inq.at[pl.ds(0, CHUNK)], gbs[b],
                                          sgs[b]).wait()
                    pltpu.async_copy(gbs[b], acc.at[dstv.at[NBUF * m + b]],
                                     sss[b], add=True)
                for b in range(NBUF):
                    pltpu.make_async_copy(inq.at[pl.ds(0, CHUNK)], gbs[b],
                                          sss[b]).wait()

                    @pl.when(m < SLAB // NBUF - 1)
                    def _(b=b):
                        pltpu.async_copy(
                            inq.at[idxv.at[NBUF * (m + 1) + b]], gbs[b],
                            sgs[b])

                return carry2

            lax.fori_loop(0, SLAB // NBUF, grp, 0)
            return carry

        lax.fori_loop(0, NCHUNK_T // SLAB, slab_body, 0)
        plsc.subcore_barrier()

        def wstep(b, carry):
            row0 = s * ROWS_T + b * WBLK
            pltpu.sync_copy(acc.at[pl.ds(row0, WBLK)],
                            out.at[c, pl.ds(row0, WBLK)])
            return carry

        lax.fori_loop(0, ROWS_T // WBLK, wstep, 0)

    return k


def _spmm_es_kernel(W):
    mesh = plsc.VectorSubcoreMesh(core_axis_name="c", subcore_axis_name="s")
    nch = NCHUNK_T // 2

    @functools.partial(
        pl.kernel,
        out_type=jax.ShapeDtypeStruct((2, ACC_ROWS, W), jnp.float32),
        mesh=mesh,
        scratch_types=[
            pltpu.VMEM_SHARED((ACC_ROWS, W), jnp.float32),
            pltpu.VMEM((nch, CHUNK), jnp.int32),
            pltpu.VMEM((nch, CHUNK), jnp.int32),
        ] + [pltpu.VMEM((CHUNK, W), jnp.float32)] * NBUF
          + [pltpu.SemaphoreType.DMA] * (2 * NBUF),
        compiler_params=_SC_PARAMS,
    )
    def k(inq, idx, dstl, zrows, out, acc, idxv, dstv,
          gb0, gb1, gb2, gb3, sg0, sg1, sg2, sg3, ss0, ss1, ss2, ss3):
        gbs = (gb0, gb1, gb2, gb3)
        sgs = (sg0, sg1, sg2, sg3)
        sss = (ss0, ss1, ss2, ss3)
        c = lax.axis_index("c")
        s = lax.axis_index("s")
        pltpu.sync_copy(idx.at[c, s], idxv)
        pltpu.sync_copy(dstl.at[c, s], dstv)
        pltpu.sync_copy(zrows, acc.at[pl.ds(s * ROWS_T, ROWS_T)])
        plsc.subcore_barrier()

        for b in range(NBUF):
            pltpu.async_copy(inq.at[idxv.at[b]], gbs[b], sgs[b])

        def grp(m, carry):
            for b in range(NBUF):
                pltpu.make_async_copy(inq.at[pl.ds(0, CHUNK)], gbs[b],
                                      sgs[b]).wait()
                pltpu.async_copy(gbs[b], acc.at[dstv.at[NBUF * m + b]],
                                 sss[b], add=True)
            for b in range(NBUF):
                pltpu.make_async_copy(inq.at[pl.ds(0, CHUNK)], gbs[b],
                                      sss[b]).wait()

                @pl.when(m < nch // NBUF - 1)
                def _(b=b):
                    pltpu.async_copy(inq.at[idxv.at[NBUF * (m + 1) + b]],
                                     gbs[b], sgs[b])

            return carry

        lax.fori_loop(0, nch // NBUF, grp, 0)
        plsc.subcore_barrier()

        def wstep(b, carry):
            row0 = s * ROWS_T + b * WBLK
            pltpu.sync_copy(acc.at[pl.ds(row0, WBLK)],
                            out.at[c, pl.ds(row0, WBLK)])
            return carry

        lax.fori_loop(0, ROWS_T // WBLK, wstep, 0)

    return k


def _deg_kernel():
    mesh = plsc.VectorSubcoreMesh(core_axis_name="c", subcore_axis_name="s")

    @functools.partial(
        pl.kernel,
        out_type=jax.ShapeDtypeStruct((2, ACC_ROWS, 16), jnp.float32),
        mesh=mesh,
        scratch_types=[
            pltpu.VMEM_SHARED((ACC_ROWS, 16), jnp.float32),
            pltpu.VMEM((NCHUNK_T // 2, CHUNK), jnp.int32),
            pltpu.VMEM((CHUNK, 16), jnp.float32),
        ] + [pltpu.SemaphoreType.DMA] * NBUF,
        compiler_params=_SC_PARAMS,
    )
    def k(dstl, ones_h, zrows, out, acc, dstv, onesv, ss0, ss1, ss2, ss3):
        sss = (ss0, ss1, ss2, ss3)
        c = lax.axis_index("c")
        s = lax.axis_index("s")
        pltpu.sync_copy(dstl.at[c, s], dstv)
        pltpu.sync_copy(ones_h, onesv)
        pltpu.sync_copy(zrows, acc.at[pl.ds(s * ROWS_T, ROWS_T)])
        plsc.subcore_barrier()

        for b in range(NBUF):
            pltpu.async_copy(onesv, acc.at[dstv.at[b]], sss[b], add=True)

        nch = NCHUNK_T // 2

        def grp(m, carry):
            for b in range(NBUF):
                pltpu.make_async_copy(ones_h, onesv, sss[b]).wait()

                @pl.when(m < nch // NBUF - 1)
                def _(b=b):
                    pltpu.async_copy(onesv, acc.at[dstv.at[NBUF * (m + 1) + b]],
                                     sss[b], add=True)

            return carry

        lax.fori_loop(0, nch // NBUF, grp, 0)
        plsc.subcore_barrier()

        def wstep(b, carry):
            row0 = s * ROWS_T + b * WBLK
            pltpu.sync_copy(acc.at[pl.ds(row0, WBLK)],
                            out.at[c, pl.ds(row0, WBLK)])
            return carry

        lax.fori_loop(0, ROWS_T // WBLK, wstep, 0)

    return k


def _norm_of(degblk):
    cnt = degblk[0, :, 0:1] + degblk[1, :, 0:1]
    return lax.rsqrt(cnt + 1.0)


NB = 2000


def _idx_body(src_ref, o0_ref, o1_ref, o2_ref, o3_ref, o4_ref, o5_ref):
    src = src_ref[...]
    cofs = lax.broadcasted_iota(jnp.int32, (2, EPAD), 0)
    outs = (o0_ref, o1_ref, o2_ref, o3_ref, o4_ref, o5_ref)
    for l, o_ref in enumerate(outs):
        if l < 2:
            o_ref[...] = 4 * src + 2 * l + cofs
        else:
            o_ref[...] = 8 * src + 2 * (l - 2) + cofs


def _router_in_body(x_ref, w_ref, b_ref, o_ref):
    o_ref[...] = jax.nn.relu(
        jnp.dot(x_ref[...], w_ref[...], preferred_element_type=jnp.float32)
        + b_ref[...])


def _gin_body(h_ref, agg_ref, w1_ref, b1_ref, w2_ref, b2_ref, o_ref):
    h = h_ref[...] + agg_ref[0] + agg_ref[1]
    h = jax.nn.relu(jnp.dot(h, w1_ref[...], preferred_element_type=jnp.float32)
                    + b1_ref[...])
    o_ref[...] = jax.nn.relu(
        jnp.dot(h, w2_ref[...], preferred_element_type=jnp.float32)
        + b2_ref[...])


def _router_head_body(h_ref, batch_ref, wo_ref, bo_ref, o_ref):
    onehot = (lax.broadcasted_iota(jnp.int32, (N, NG), 1)
              == batch_ref[...]).astype(jnp.float32)
    pooled = lax.dot_general(onehot, h_ref[...], (((0,), (0,)), ((), ())),
                             preferred_element_type=jnp.float32)
    counts = jnp.maximum(jnp.sum(onehot, axis=0, keepdims=True), 1.0).T
    logits = (jnp.dot(pooled / counts, wo_ref[...],
                      preferred_element_type=jnp.float32) + bo_ref[...])
    m = jnp.max(logits, axis=-1, keepdims=True)
    e = jnp.exp(logits - m)
    w = e / jnp.sum(e, axis=-1, keepdims=True)
    o_ref[...] = w / counts


def _prep_u_body(x_ref, f0_ref, f1_ref, f2_ref, deg_ref, o_ref):
    nrm = _norm_of(deg_ref[...])
    o_ref[...] = jnp.concatenate(
        [x_ref[...], f0_ref[:, D:], f1_ref[:, D:], f2_ref[:, D:],
         jnp.zeros((x_ref.shape[0], 32), jnp.float32)], axis=-1) * nrm


def _z1_body(p0_ref, p1_ref, u_ref, deg_ref, w1_ref, b1_ref, o_ref):
    nrm = _norm_of(deg_ref[...])
    pcat = jnp.concatenate([p0_ref[0], p0_ref[1], p1_ref[0], p1_ref[1]],
                           axis=-1)
    sfull = nrm * (pcat + u_ref[...])
    sx = sfull[:, :D]
    sf = sfull[:, D:D + 3 * ENC]
    shared = jnp.dot(sx, w1_ref[:D], preferred_element_type=jnp.float32)
    cols = []
    for i in range(NUM_ENC):
        z = jax.nn.relu(
            shared
            + jnp.dot(sf[:, ENC * i:ENC * (i + 1)], w1_ref[D:],
                      preferred_element_type=jnp.float32)
            + b1_ref[...])
        cols.append(z * nrm)
    o_ref[...] = jnp.concatenate(cols, axis=-1)


def _z2_body(a0_ref, a1_ref, a2_ref, a3_ref, u_ref, deg_ref, w2_ref, b2_ref,
             w3_ref, o_ref):
    nrm = _norm_of(deg_ref[...])
    acat = jnp.concatenate(
        [a0_ref[0], a0_ref[1], a1_ref[0], a1_ref[1],
         a2_ref[0], a2_ref[1], a3_ref[0], a3_ref[1]], axis=-1)
    cols = []
    for i in range(NUM_ENC):
        sfull = nrm * (acat[:, HID * i:HID * (i + 1)]
                       + u_ref[:, HID * i:HID * (i + 1)])
        t = jax.nn.relu(
            jnp.dot(sfull, w2_ref[...], preferred_element_type=jnp.float32)
            + b2_ref[...])
        y = jnp.dot(t, w3_ref[...], preferred_element_type=jnp.float32)
        cols.append(y * nrm)
    o_ref[...] = jnp.concatenate(cols, axis=-1)


def _final_body(a0_ref, a1_ref, y_ref, deg_ref, b3_ref, batch_ref,
                wc_ref, o_ref, acc_ref):
    i = pl.program_id(0)

    @pl.when(i == 0)
    def _():
        acc_ref[...] = jnp.zeros_like(acc_ref)

    nrm = _norm_of(deg_ref[...])
    onehot = (lax.broadcasted_iota(jnp.int32, (NB, NG), 1)
              == batch_ref[...]).astype(jnp.float32)
    acat = jnp.concatenate([a0_ref[0], a0_ref[1], a1_ref[0], a1_ref[1]],
                           axis=-1)
    for e in range(NUM_ENC):
        z3 = jax.nn.relu(nrm * (acat[:, OUT * e:OUT * (e + 1)]
                                + y_ref[:, OUT * e:OUT * (e + 1)])
                         + b3_ref[...])
        acc_ref[:, OUT * e:OUT * (e + 1)] += lax.dot_general(
            onehot, z3, (((0,), (0,)), ((), ())),
            preferred_element_type=jnp.float32)

    @pl.when(i == pl.num_programs(0) - 1)
    def _():
        o_ref[...] = sum(
            wc_ref[:, e:e + 1] * acc_ref[:, OUT * e:OUT * (e + 1)]
            for e in range(NUM_ENC))


def _row_grid(nblk):
    return (N // nblk,)


def kernel(x, edge_index, batch, enc0, enc1, enc2, Wr_in, br_in, Wg1, bg1,
           Wg2, bg2, Wr_out, br_out, W1, b1, W2, b2, W3, b3):
    src = edge_index[0]
    dst = edge_index[1]
    npad = EPAD - E
    src_pad = jnp.concatenate([src, jnp.zeros((npad,), jnp.int32)])
    dst_pad = jnp.concatenate([dst, jnp.full((npad,), N, jnp.int32)])

    idxp = pl.pallas_call(
        _idx_body,
        in_specs=[pl.BlockSpec((1, EPAD), lambda: (0, 0))],
        out_specs=[pl.BlockSpec((2, EPAD), lambda: (0, 0))] * 6,
        out_shape=[jax.ShapeDtypeStruct((2, EPAD), jnp.int32)] * 6,
    )(src_pad.reshape(1, EPAD))
    idxp = [a.reshape(2, 16, NCHUNK_T, CHUNK) for a in idxp]
    idx4 = idxp[0:2]
    idx8 = idxp[2:6]
    src232 = src_pad.reshape(2, 16, NCHUNK_T // 2, CHUNK)
    dst16 = dst_pad.reshape(16, NCHUNK_T, CHUNK)
    dst232 = dst_pad.reshape(2, 16, NCHUNK_T // 2, CHUNK)

    z16 = jnp.zeros((ROWS_T, 16), jnp.float32)
    z64 = jnp.zeros((ROWS_T, 64), jnp.float32)
    z96 = jnp.zeros((ROWS_T, 96), jnp.float32)
    ones16 = jnp.ones((CHUNK, 16), jnp.float32)

    deg = _deg_kernel()(dst232, ones16, z16)

    h = pl.pallas_call(
        _router_in_body,
        grid=_row_grid(NB),
        in_specs=[
            pl.BlockSpec((NB, D), lambda i: (i, 0)),
            pl.BlockSpec((D, H), lambda i: (0, 0)),
            pl.BlockSpec((1, H), lambda i: (0, 0)),
        ],
        out_specs=pl.BlockSpec((NB, H), lambda i: (i, 0)),
        out_shape=jax.ShapeDtypeStruct((N, H), jnp.float32),
    )(x, Wr_in, br_in.reshape(1, H))

    spmm64 = _spmm_es_kernel(64)
    for l in range(DEPTH):
        agg = spmm64(h, src232, dst232, z64)
        h = pl.pallas_call(
            _gin_body,
            grid=_row_grid(NB),
            in_specs=[
                pl.BlockSpec((NB, H), lambda i: (i, 0)),
                pl.BlockSpec((2, NB, H), lambda i: (0, i, 0)),
                pl.BlockSpec((H, H), lambda i: (0, 0)),
                pl.BlockSpec((1, H), lambda i: (0, 0)),
                pl.BlockSpec((H, H), lambda i: (0, 0)),
                pl.BlockSpec((1, H), lambda i: (0, 0)),
            ],
            out_specs=pl.BlockSpec((NB, H), lambda i: (i, 0)),
            out_shape=jax.ShapeDtypeStruct((N, H), jnp.float32),
        )(h, agg, Wg1[l], bg1[l].reshape(1, H), Wg2[l], bg2[l].reshape(1, H))

    batch_col = batch.astype(jnp.int32).reshape(N, 1)
    wc = pl.pallas_call(
        _router_head_body,
        in_specs=[
            pl.BlockSpec((N, H), lambda: (0, 0)),
            pl.BlockSpec((N, 1), lambda: (0, 0)),
            pl.BlockSpec((H, NUM_ENC), lambda: (0, 0)),
            pl.BlockSpec((1, NUM_ENC), lambda: (0, 0)),
        ],
        out_specs=pl.BlockSpec((NG, NUM_ENC), lambda: (0, 0)),
        out_shape=jax.ShapeDtypeStruct((NG, NUM_ENC), jnp.float32),
    )(h, batch_col, Wr_out, br_out.reshape(1, NUM_ENC))

    u = pl.pallas_call(
        _prep_u_body,
        grid=_row_grid(NB),
        in_specs=[
            pl.BlockSpec((NB, D), lambda i: (i, 0)),
            pl.BlockSpec((NB, D + ENC), lambda i: (i, 0)),
            pl.BlockSpec((NB, D + ENC), lambda i: (i, 0)),
            pl.BlockSpec((NB, D + ENC), lambda i: (i, 0)),
            pl.BlockSpec((2, NB, 16), lambda i: (0, i, 0)),
        ],
        out_specs=pl.BlockSpec((NB, 384), lambda i: (i, 0)),
        out_shape=jax.ShapeDtypeStruct((N, 384), jnp.float32),
    )(x, enc0, enc1, enc2, deg)

    spmm96 = _spmm_kernel(96)
    u4 = u.reshape(4 * N, 96)
    p = [spmm96(u4, idx4[i], dst16, z96) for i in range(2)]

    uz = pl.pallas_call(
        _z1_body,
        grid=_row_grid(NB),
        in_specs=[
            pl.BlockSpec((2, NB, 96), lambda i: (0, i, 0)),
            pl.BlockSpec((2, NB, 96), lambda i: (0, i, 0)),
            pl.BlockSpec((NB, 384), lambda i: (i, 0)),
            pl.BlockSpec((2, NB, 16), lambda i: (0, i, 0)),
            pl.BlockSpec((D + ENC, HID), lambda i: (0, 0)),
            pl.BlockSpec((1, HID), lambda i: (0, 0)),
        ],
        out_specs=pl.BlockSpec((NB, 3 * HID), lambda i: (i, 0)),
        out_shape=jax.ShapeDtypeStruct((N, 3 * HID), jnp.float32),
    )(p[0], p[1], u, deg, W1, b1.reshape(1, HID))

    uz8 = uz.reshape(8 * N, 96)
    a2 = [spmm96(uz8, idx8[i], dst16, z96) for i in range(4)]

    y = pl.pallas_call(
        _z2_body,
        grid=_row_grid(NB),
        in_specs=[
            pl.BlockSpec((2, NB, 96), lambda i: (0, i, 0)),
            pl.BlockSpec((2, NB, 96), lambda i: (0, i, 0)),
            pl.BlockSpec((2, NB, 96), lambda i: (0, i, 0)),
            pl.BlockSpec((2, NB, 96), lambda i: (0, i, 0)),
            pl.BlockSpec((NB, 3 * HID), lambda i: (i, 0)),
            pl.BlockSpec((2, NB, 16), lambda i: (0, i, 0)),
            pl.BlockSpec((HID, HID), lambda i: (0, 0)),
            pl.BlockSpec((1, HID), lambda i: (0, 0)),
            pl.BlockSpec((HID, OUT), lambda i: (0, 0)),
        ],
        out_specs=pl.BlockSpec((NB, 3 * OUT), lambda i: (i, 0)),
        out_shape=jax.ShapeDtypeStruct((N, 3 * OUT), jnp.float32),
    )(a2[0], a2[1], a2[2], a2[3], uz, deg, W2, b2.reshape(1, HID), W3)

    y4 = y.reshape(4 * N, 96)
    a3 = [spmm96(y4, idx4[i], dst16, z96) for i in range(2)]

    final = pl.pallas_call(
        _final_body,
        grid=_row_grid(NB),
        in_specs=[
            pl.BlockSpec((2, NB, 96), lambda i: (0, i, 0)),
            pl.BlockSpec((2, NB, 96), lambda i: (0, i, 0)),
            pl.BlockSpec((NB, 3 * OUT), lambda i: (i, 0)),
            pl.BlockSpec((2, NB, 16), lambda i: (0, i, 0)),
            pl.BlockSpec((1, OUT), lambda i: (0, 0)),
            pl.BlockSpec((NB, 1), lambda i: (i, 0)),
            pl.BlockSpec((NG, NUM_ENC), lambda i: (0, 0)),
        ],
        out_specs=pl.BlockSpec((NG, OUT), lambda i: (0, 0)),
        out_shape=jax.ShapeDtypeStruct((NG, OUT), jnp.float32),
        scratch_shapes=[pltpu.VMEM((NG, 3 * OUT), jnp.float32)],
    )(a3[0], a3[1], y, deg, b3.reshape(1, OUT), batch_col, wc)

    return final


# speedup vs baseline: 1.0401x; 1.0160x over previous
import functools

import jax
import jax.numpy as jnp
from jax import lax
from jax.experimental import pallas as pl
from jax.experimental.pallas import tpu as pltpu
from jax.experimental.pallas import tpu_sc as plsc

N = 10000
E = 160000
D = 256
ENC = 32
NG = 16
H = 64
DEPTH = 4
HID = 256
OUT = 128
NUM_ENC = 3

CHUNK = 128
EPAD = 163840
NCHUNK_T = 80
SLAB = 40
NBUF = 4
ROWS_T = 640
ACC_ROWS = 10240
WBLK = 80

_SC_PARAMS = pltpu.CompilerParams(use_tc_tiling_on_sc=False)


def _spmm_kernel(Fh):
    mesh = plsc.VectorSubcoreMesh(core_axis_name="c", subcore_axis_name="s")

    @functools.partial(
        pl.kernel,
        out_type=jax.ShapeDtypeStruct((2, ACC_ROWS, Fh), jnp.float32),
        mesh=mesh,
        scratch_types=[
            pltpu.VMEM_SHARED((ACC_ROWS, Fh), jnp.float32),
            pltpu.VMEM((SLAB, CHUNK), jnp.int32),
            pltpu.VMEM((SLAB, CHUNK), jnp.int32),
        ] + [pltpu.VMEM((CHUNK, Fh), jnp.float32)] * NBUF
          + [pltpu.SemaphoreType.DMA] * (2 * NBUF),
        compiler_params=_SC_PARAMS,
    )
    def k(inq, idx, dstl, zrows, out, acc, idxv, dstv,
          gb0, gb1, gb2, gb3, sg0, sg1, sg2, sg3, ss0, ss1, ss2, ss3):
        gbs = (gb0, gb1, gb2, gb3)
        sgs = (sg0, sg1, sg2, sg3)
        sss = (ss0, ss1, ss2, ss3)
        c = lax.axis_index("c")
        s = lax.axis_index("s")
        pltpu.sync_copy(zrows, acc.at[pl.ds(s * ROWS_T, ROWS_T)])
        plsc.subcore_barrier()

        def slab_body(sl, carry):
            pltpu.sync_copy(idx.at[c, s, pl.ds(sl * SLAB, SLAB)], idxv)
            pltpu.sync_copy(dstl.at[s, pl.ds(sl * SLAB, SLAB)], dstv)
            for b in range(NBUF):
                pltpu.async_copy(inq.at[idxv.at[b]], gbs[b], sgs[b])

            def grp(m, carry2):
                for b in range(NBUF):
                    pltpu.make_async_copy(inq.at[pl.ds(0, CHUNK)], gbs[b],
                                          sgs[b]).wait()
                    pltpu.async_copy(gbs[b], acc.at[dstv.at[NBUF * m + b]],
                                     sss[b], add=True)
                for b in range(NBUF):
                    pltpu.make_async_copy(inq.at[pl.ds(0, CHUNK)], gbs[b],
                                          sss[b]).wait()

                    @pl.when(m < SLAB // NBUF - 1)
                    def _(b=b):
                        pltpu.async_copy(
                            inq.at[idxv.at[NBUF * (m + 1) + b]], gbs[b],
                            sgs[b])

                return carry2

            lax.fori_loop(0, SLAB // NBUF, grp, 0)
            return carry

        lax.fori_loop(0, NCHUNK_T // SLAB, slab_body, 0)
        plsc.subcore_barrier()

        def wstep(b, carry):
            row0 = s * ROWS_T + b * WBLK
            pltpu.sync_copy(acc.at[pl.ds(row0, WBLK)],
                            out.at[c, pl.ds(row0, WBLK)])
            return carry

        lax.fori_loop(0, ROWS_T // WBLK, wstep, 0)

    return k


def _spmm_es_kernel(W):
    mesh = plsc.VectorSubcoreMesh(core_axis_name="c", subcore_axis_name="s")
    nch = NCHUNK_T // 2

    @functools.partial(
        pl.kernel,
        out_type=jax.ShapeDtypeStruct((2, ACC_ROWS, W), jnp.float32),
        mesh=mesh,
        scratch_types=[
            pltpu.VMEM_SHARED((ACC_ROWS, W), jnp.float32),
            pltpu.VMEM((nch, CHUNK), jnp.int32),
            pltpu.VMEM((nch, CHUNK), jnp.int32),
        ] + [pltpu.VMEM((CHUNK, W), jnp.float32)] * NBUF
          + [pltpu.SemaphoreType.DMA] * (2 * NBUF),
        compiler_params=_SC_PARAMS,
    )
    def k(inq, idx, dstl, zrows, out, acc, idxv, dstv,
          gb0, gb1, gb2, gb3, sg0, sg1, sg2, sg3, ss0, ss1, ss2, ss3):
        gbs = (gb0, gb1, gb2, gb3)
        sgs = (sg0, sg1, sg2, sg3)
        sss = (ss0, ss1, ss2, ss3)
        c = lax.axis_index("c")
        s = lax.axis_index("s")
        pltpu.sync_copy(idx.at[c, s], idxv)
        pltpu.sync_copy(dstl.at[c, s], dstv)
        pltpu.sync_copy(zrows, acc.at[pl.ds(s * ROWS_T, ROWS_T)])
        plsc.subcore_barrier()

        for b in range(NBUF):
            pltpu.async_copy(inq.at[idxv.at[b]], gbs[b], sgs[b])

        def grp(m, carry):
            for b in range(NBUF):
                pltpu.make_async_copy(inq.at[pl.ds(0, CHUNK)], gbs[b],
                                      sgs[b]).wait()
                pltpu.async_copy(gbs[b], acc.at[dstv.at[NBUF * m + b]],
                                 sss[b], add=True)
            for b in range(NBUF):
                pltpu.make_async_copy(inq.at[pl.ds(0, CHUNK)], gbs[b],
                                      sss[b]).wait()

                @pl.when(m < nch // NBUF - 1)
                def _(b=b):
                    pltpu.async_copy(inq.at[idxv.at[NBUF * (m + 1) + b]],
                                     gbs[b], sgs[b])

            return carry

        lax.fori_loop(0, nch // NBUF, grp, 0)
        plsc.subcore_barrier()

        def wstep(b, carry):
            row0 = s * ROWS_T + b * WBLK
            pltpu.sync_copy(acc.at[pl.ds(row0, WBLK)],
                            out.at[c, pl.ds(row0, WBLK)])
            return carry

        lax.fori_loop(0, ROWS_T // WBLK, wstep, 0)

    return k


def _deg_kernel():
    mesh = plsc.VectorSubcoreMesh(core_axis_name="c", subcore_axis_name="s")

    @functools.partial(
        pl.kernel,
        out_type=jax.ShapeDtypeStruct((2, ACC_ROWS, 16), jnp.float32),
        mesh=mesh,
        scratch_types=[
            pltpu.VMEM_SHARED((ACC_ROWS, 16), jnp.float32),
            pltpu.VMEM((NCHUNK_T // 2, CHUNK), jnp.int32),
            pltpu.VMEM((CHUNK, 16), jnp.float32),
        ] + [pltpu.SemaphoreType.DMA] * NBUF,
        compiler_params=_SC_PARAMS,
    )
    def k(dstl, ones_h, zrows, out, acc, dstv, onesv, ss0, ss1, ss2, ss3):
        sss = (ss0, ss1, ss2, ss3)
        c = lax.axis_index("c")
        s = lax.axis_index("s")
        pltpu.sync_copy(dstl.at[c, s], dstv)
        pltpu.sync_copy(ones_h, onesv)
        pltpu.sync_copy(zrows, acc.at[pl.ds(s * ROWS_T, ROWS_T)])
        plsc.subcore_barrier()

        for b in range(NBUF):
            pltpu.async_copy(onesv, acc.at[dstv.at[b]], sss[b], add=True)

        nch = NCHUNK_T // 2

        def grp(m, carry):
            for b in range(NBUF):
                pltpu.make_async_copy(ones_h, onesv, sss[b]).wait()

                @pl.when(m < nch // NBUF - 1)
                def _(b=b):
                    pltpu.async_copy(onesv, acc.at[dstv.at[NBUF * (m + 1) + b]],
                                     sss[b], add=True)

            return carry

        lax.fori_loop(0, nch // NBUF, grp, 0)
        plsc.subcore_barrier()

        def wstep(b, carry):
            row0 = s * ROWS_T + b * WBLK
            pltpu.sync_copy(acc.at[pl.ds(row0, WBLK)],
                            out.at[c, pl.ds(row0, WBLK)])
            return carry

        lax.fori_loop(0, ROWS_T // WBLK, wstep, 0)

    return k


def _norm_of(degblk):
    cnt = degblk[0, :, 0:1] + degblk[1, :, 0:1]
    return lax.rsqrt(cnt + 1.0)


NB = 2000


def _idx_body(edge_ref, osrc_ref, odst_ref, o0_ref, o1_ref, o2_ref, o3_ref,
              o4_ref, o5_ref):
    pad = jnp.zeros((1, EPAD - E), jnp.int32)
    src = jnp.concatenate([edge_ref[0:1], pad], axis=-1)
    osrc_ref[...] = src
    odst_ref[...] = jnp.concatenate([edge_ref[1:2], pad + N], axis=-1)
    cofs = lax.broadcasted_iota(jnp.int32, (2, EPAD), 0)
    outs = (o0_ref, o1_ref, o2_ref, o3_ref, o4_ref, o5_ref)
    for l, o_ref in enumerate(outs):
        if l < 2:
            o_ref[...] = 4 * src + 2 * l + cofs
        else:
            o_ref[...] = 8 * src + 2 * (l - 2) + cofs


def _router_in_body(x_ref, w_ref, b_ref, o_ref):
    o_ref[...] = jax.nn.relu(
        jnp.dot(x_ref[...], w_ref[...], preferred_element_type=jnp.float32)
        + b_ref[...])


def _gin_body(h_ref, agg_ref, w1_ref, b1_ref, w2_ref, b2_ref, o_ref):
    h = h_ref[...] + agg_ref[0] + agg_ref[1]
    h = jax.nn.relu(jnp.dot(h, w1_ref[...], preferred_element_type=jnp.float32)
                    + b1_ref[...])
    o_ref[...] = jax.nn.relu(
        jnp.dot(h, w2_ref[...], preferred_element_type=jnp.float32)
        + b2_ref[...])


def _router_head_body(h_ref, batch_ref, wo_ref, bo_ref, o_ref):
    onehot = (lax.broadcasted_iota(jnp.int32, (N, NG), 1)
              == batch_ref[...]).astype(jnp.float32)
    pooled = lax.dot_general(onehot, h_ref[...], (((0,), (0,)), ((), ())),
                             preferred_element_type=jnp.float32)
    counts = jnp.maximum(jnp.sum(onehot, axis=0, keepdims=True), 1.0).T
    logits = (jnp.dot(pooled / counts, wo_ref[...],
                      preferred_element_type=jnp.float32) + bo_ref[...])
    m = jnp.max(logits, axis=-1, keepdims=True)
    e = jnp.exp(logits - m)
    w = e / jnp.sum(e, axis=-1, keepdims=True)
    o_ref[...] = w / counts


def _prep_u_body(x_ref, f0_ref, f1_ref, f2_ref, deg_ref, o_ref):
    nrm = _norm_of(deg_ref[...])
    o_ref[...] = jnp.concatenate(
        [x_ref[...], f0_ref[:, D:], f1_ref[:, D:], f2_ref[:, D:],
         jnp.zeros((x_ref.shape[0], 32), jnp.float32)], axis=-1) * nrm


def _z1_body(p0_ref, p1_ref, u_ref, deg_ref, w1_ref, b1_ref, o_ref):
    nrm = _norm_of(deg_ref[...])
    pcat = jnp.concatenate([p0_ref[0], p0_ref[1], p1_ref[0], p1_ref[1]],
                           axis=-1)
    sfull = nrm * (pcat + u_ref[...])
    sx = sfull[:, :D]
    sf = sfull[:, D:D + 3 * ENC]
    shared = jnp.dot(sx, w1_ref[:D], preferred_element_type=jnp.float32)
    cols = []
    for i in range(NUM_ENC):
        z = jax.nn.relu(
            shared
            + jnp.dot(sf[:, ENC * i:ENC * (i + 1)], w1_ref[D:],
                      preferred_element_type=jnp.float32)
            + b1_ref[...])
        cols.append(z * nrm)
    o_ref[...] = jnp.concatenate(cols, axis=-1)


def _z2_body(a0_ref, a1_ref, a2_ref, a3_ref, u_ref, deg_ref, w2_ref, b2_ref,
             w3_ref, o_ref):
    nrm = _norm_of(deg_ref[...])
    acat = jnp.concatenate(
        [a0_ref[0], a0_ref[1], a1_ref[0], a1_ref[1],
         a2_ref[0], a2_ref[1], a3_ref[0], a3_ref[1]], axis=-1)
    cols = []
    for i in range(NUM_ENC):
        sfull = nrm * (acat[:, HID * i:HID * (i + 1)]
                       + u_ref[:, HID * i:HID * (i + 1)])
        t = jax.nn.relu(
            jnp.dot(sfull, w2_ref[...], preferred_element_type=jnp.float32)
            + b2_ref[...])
        y = jnp.dot(t, w3_ref[...], preferred_element_type=jnp.float32)
        cols.append(y * nrm)
    o_ref[...] = jnp.concatenate(cols, axis=-1)


def _final_body(a0_ref, a1_ref, y_ref, deg_ref, b3_ref, batch_ref,
                wc_ref, o_ref, acc_ref):
    i = pl.program_id(0)

    @pl.when(i == 0)
    def _():
        acc_ref[...] = jnp.zeros_like(acc_ref)

    nrm = _norm_of(deg_ref[...])
    onehot = (lax.broadcasted_iota(jnp.int32, (NB, NG), 1)
              == batch_ref[...]).astype(jnp.float32)
    acat = jnp.concatenate([a0_ref[0], a0_ref[1], a1_ref[0], a1_ref[1]],
                           axis=-1)
    for e in range(NUM_ENC):
        z3 = jax.nn.relu(nrm * (acat[:, OUT * e:OUT * (e + 1)]
                                + y_ref[:, OUT * e:OUT * (e + 1)])
                         + b3_ref[...])
        acc_ref[:, OUT * e:OUT * (e + 1)] += lax.dot_general(
            onehot, z3, (((0,), (0,)), ((), ())),
            preferred_element_type=jnp.float32)

    @pl.when(i == pl.num_programs(0) - 1)
    def _():
        o_ref[...] = sum(
            wc_ref[:, e:e + 1] * acc_ref[:, OUT * e:OUT * (e + 1)]
            for e in range(NUM_ENC))


def _row_grid(nblk):
    return (N // nblk,)


def kernel(x, edge_index, batch, enc0, enc1, enc2, Wr_in, br_in, Wg1, bg1,
           Wg2, bg2, Wr_out, br_out, W1, b1, W2, b2, W3, b3):
    outs = pl.pallas_call(
        _idx_body,
        in_specs=[pl.BlockSpec((2, E), lambda: (0, 0))],
        out_specs=([pl.BlockSpec((1, EPAD), lambda: (0, 0))] * 2
                   + [pl.BlockSpec((2, EPAD), lambda: (0, 0))] * 6),
        out_shape=([jax.ShapeDtypeStruct((1, EPAD), jnp.int32)] * 2
                   + [jax.ShapeDtypeStruct((2, EPAD), jnp.int32)] * 6),
    )(edge_index)
    src_pad, dst_pad = outs[0], outs[1]
    idxp = [a.reshape(2, 16, NCHUNK_T, CHUNK) for a in outs[2:]]
    idx4 = idxp[0:2]
    idx8 = idxp[2:6]
    src232 = src_pad.reshape(2, 16, NCHUNK_T // 2, CHUNK)
    dst16 = dst_pad.reshape(16, NCHUNK_T, CHUNK)
    dst232 = dst_pad.reshape(2, 16, NCHUNK_T // 2, CHUNK)

    z16 = jnp.zeros((ROWS_T, 16), jnp.float32)
    z64 = jnp.zeros((ROWS_T, 64), jnp.float32)
    z96 = jnp.zeros((ROWS_T, 96), jnp.float32)
    ones16 = jnp.ones((CHUNK, 16), jnp.float32)

    deg = _deg_kernel()(dst232, ones16, z16)

    h = pl.pallas_call(
        _router_in_body,
        grid=_row_grid(NB),
        in_specs=[
            pl.BlockSpec((NB, D), lambda i: (i, 0)),
            pl.BlockSpec((D, H), lambda i: (0, 0)),
            pl.BlockSpec((1, H), lambda i: (0, 0)),
        ],
        out_specs=pl.BlockSpec((NB, H), lambda i: (i, 0)),
        out_shape=jax.ShapeDtypeStruct((N, H), jnp.float32),
    )(x, Wr_in, br_in.reshape(1, H))

    spmm64 = _spmm_es_kernel(64)
    for l in range(DEPTH):
        agg = spmm64(h, src232, dst232, z64)
        h = pl.pallas_call(
            _gin_body,
            grid=_row_grid(NB),
            in_specs=[
                pl.BlockSpec((NB, H), lambda i: (i, 0)),
                pl.BlockSpec((2, NB, H), lambda i: (0, i, 0)),
                pl.BlockSpec((H, H), lambda i: (0, 0)),
                pl.BlockSpec((1, H), lambda i: (0, 0)),
                pl.BlockSpec((H, H), lambda i: (0, 0)),
                pl.BlockSpec((1, H), lambda i: (0, 0)),
            ],
            out_specs=pl.BlockSpec((NB, H), lambda i: (i, 0)),
            out_shape=jax.ShapeDtypeStruct((N, H), jnp.float32),
        )(h, agg, Wg1[l], bg1[l].reshape(1, H), Wg2[l], bg2[l].reshape(1, H))

    batch_col = batch.astype(jnp.int32).reshape(N, 1)
    wc = pl.pallas_call(
        _router_head_body,
        in_specs=[
            pl.BlockSpec((N, H), lambda: (0, 0)),
            pl.BlockSpec((N, 1), lambda: (0, 0)),
            pl.BlockSpec((H, NUM_ENC), lambda: (0, 0)),
            pl.BlockSpec((1, NUM_ENC), lambda: (0, 0)),
        ],
        out_specs=pl.BlockSpec((NG, NUM_ENC), lambda: (0, 0)),
        out_shape=jax.ShapeDtypeStruct((NG, NUM_ENC), jnp.float32),
    )(h, batch_col, Wr_out, br_out.reshape(1, NUM_ENC))

    u = pl.pallas_call(
        _prep_u_body,
        grid=_row_grid(NB),
        in_specs=[
            pl.BlockSpec((NB, D), lambda i: (i, 0)),
            pl.BlockSpec((NB, D + ENC), lambda i: (i, 0)),
            pl.BlockSpec((NB, D + ENC), lambda i: (i, 0)),
            pl.BlockSpec((NB, D + ENC), lambda i: (i, 0)),
            pl.BlockSpec((2, NB, 16), lambda i: (0, i, 0)),
        ],
        out_specs=pl.BlockSpec((NB, 384), lambda i: (i, 0)),
        out_shape=jax.ShapeDtypeStruct((N, 384), jnp.float32),
    )(x, enc0, enc1, enc2, deg)

    spmm96 = _spmm_kernel(96)
    u4 = u.reshape(4 * N, 96)
    p = [spmm96(u4, idx4[i], dst16, z96) for i in range(2)]

    uz = pl.pallas_call(
        _z1_body,
        grid=_row_grid(NB),
        in_specs=[
            pl.BlockSpec((2, NB, 96), lambda i: (0, i, 0)),
            pl.BlockSpec((2, NB, 96), lambda i: (0, i, 0)),
            pl.BlockSpec((NB, 384), lambda i: (i, 0)),
            pl.BlockSpec((2, NB, 16), lambda i: (0, i, 0)),
            pl.BlockSpec((D + ENC, HID), lambda i: (0, 0)),
            pl.BlockSpec((1, HID), lambda i: (0, 0)),
        ],
        out_specs=pl.BlockSpec((NB, 3 * HID), lambda i: (i, 0)),
        out_shape=jax.ShapeDtypeStruct((N, 3 * HID), jnp.float32),
    )(p[0], p[1], u, deg, W1, b1.reshape(1, HID))

    uz8 = uz.reshape(8 * N, 96)
    a2 = [spmm96(uz8, idx8[i], dst16, z96) for i in range(4)]

    y = pl.pallas_call(
        _z2_body,
        grid=_row_grid(NB),
        in_specs=[
            pl.BlockSpec((2, NB, 96), lambda i: (0, i, 0)),
            pl.BlockSpec((2, NB, 96), lambda i: (0, i, 0)),
            pl.BlockSpec((2, NB, 96), lambda i: (0, i, 0)),
            pl.BlockSpec((2, NB, 96), lambda i: (0, i, 0)),
            pl.BlockSpec((NB, 3 * HID), lambda i: (i, 0)),
            pl.BlockSpec((2, NB, 16), lambda i: (0, i, 0)),
            pl.BlockSpec((HID, HID), lambda i: (0, 0)),
            pl.BlockSpec((1, HID), lambda i: (0, 0)),
            pl.BlockSpec((HID, OUT), lambda i: (0, 0)),
        ],
        out_specs=pl.BlockSpec((NB, 3 * OUT), lambda i: (i, 0)),
        out_shape=jax.ShapeDtypeStruct((N, 3 * OUT), jnp.float32),
    )(a2[0], a2[1], a2[2], a2[3], uz, deg, W2, b2.reshape(1, HID), W3)

    y4 = y.reshape(4 * N, 96)
    a3 = [spmm96(y4, idx4[i], dst16, z96) for i in range(2)]

    final = pl.pallas_call(
        _final_body,
        grid=_row_grid(NB),
        in_specs=[
            pl.BlockSpec((2, NB, 96), lambda i: (0, i, 0)),
            pl.BlockSpec((2, NB, 96), lambda i: (0, i, 0)),
            pl.BlockSpec((NB, 3 * OUT), lambda i: (i, 0)),
            pl.BlockSpec((2, NB, 16), lambda i: (0, i, 0)),
            pl.BlockSpec((1, OUT), lambda i: (0, 0)),
            pl.BlockSpec((NB, 1), lambda i: (i, 0)),
            pl.BlockSpec((NG, NUM_ENC), lambda i: (0, 0)),
        ],
        out_specs=pl.BlockSpec((NG, OUT), lambda i: (0, 0)),
        out_shape=jax.ShapeDtypeStruct((NG, OUT), jnp.float32),
        scratch_shapes=[pltpu.VMEM((NG, 3 * OUT), jnp.float32)],
    )(a3[0], a3[1], y, deg, b3.reshape(1, OUT), batch_col, wc)

    return final
